# baseline (speedup 1.0000x reference)
import functools

import jax
import jax.numpy as jnp
from jax import lax
from jax.experimental import pallas as pl
from jax.experimental.pallas import tpu as pltpu
from jax.experimental.pallas import tpu_sc as plsc

VOCAB = 1000000
EMBED = 32
NUM_CORES = 2
NUM_SUBCORES = 16
NUM_WORKERS = NUM_CORES * NUM_SUBCORES
TB = 128
NBUF = 4


@functools.partial(jax.jit, static_argnums=(2, 3))
def _gather_embed(tok_t, table, n_pos, n_batch):
    mesh = plsc.VectorSubcoreMesh(core_axis_name="c", subcore_axis_name="s")
    n_bblk = n_batch // TB
    assert n_bblk == NUM_WORKERS

    @functools.partial(
        pl.kernel,
        mesh=mesh,
        out_type=jax.ShapeDtypeStruct((n_pos, EMBED // 8, n_bblk, 8 * TB),
                                      jnp.float32),
        scratch_types=[
            pltpu.VMEM((n_pos, TB), jnp.int32),
        ] + [pltpu.VMEM((TB, EMBED), jnp.float32)] * NBUF
          + [pltpu.VMEM((EMBED * TB,), jnp.float32)] * NBUF
          + [pltpu.SemaphoreType.DMA] * (2 * NBUF),
        compiler_params=pltpu.CompilerParams(
            use_tc_tiling_on_sc=False, needs_layout_passes=False),
    )
    def k(tok_hbm, table_hbm, out_hbm, idxs_v, *bufs):
        rows = bufs[0:NBUF]
        tbuf = bufs[NBUF:2 * NBUF]
        sg = bufs[2 * NBUF:3 * NBUF]
        so = bufs[3 * NBUF:4 * NBUF]
        bblk = lax.axis_index("s") * NUM_CORES + lax.axis_index("c")
        b0 = bblk * TB
        pltpu.sync_copy(tok_hbm.at[:, pl.ds(b0, TB)], idxs_v)

        iot = lax.iota(jnp.int32, 16)
        bvecs = [iot + 16 * j for j in range(TB // 16)]

        def gather_start(t, rb):
            pltpu.async_copy(table_hbm.at[idxs_v.at[t]], rows[rb], sg[rb])

        def gather_wait(t, rb):
            pltpu.make_async_copy(
                table_hbm.at[idxs_v.at[t]], rows[rb], sg[rb]).wait()

        def wb_start(t, rb):
            for e8 in range(EMBED // 8):
                pltpu.async_copy(tbuf[rb].at[pl.ds(e8 * 8 * TB, 8 * TB)],
                                 out_hbm.at[t, e8, bblk], so[rb])

        def wb_wait(t, rb):
            for e8 in range(EMBED // 8):
                pltpu.make_async_copy(tbuf[rb].at[pl.ds(e8 * 8 * TB, 8 * TB)],
                                      out_hbm.at[t, e8, bblk], so[rb]).wait()

        for rb in range(NBUF):
            gather_start(rb, rb)

        def body(kk, carry):
            for rb in range(NBUF):
                t = NBUF * kk + rb
                gather_wait(t, rb)
                @pl.when(kk > 0)
                def _():
                    wb_wait(t - NBUF, rb)
                for e in range(EMBED):
                    evec = jnp.full((16,), e, jnp.int32)
                    for j in range(TB // 16):
                        v = plsc.load_gather(rows[rb], [bvecs[j], evec])
                        tbuf[rb][pl.ds(e * TB + j * 16, 16)] = v
                wb_start(t, rb)
                @pl.when(t + NBUF < n_pos)
                def _():
                    gather_start(t + NBUF, rb)
            return carry

        lax.fori_loop(0, n_pos // NBUF, body, 0)
        for rb in range(NBUF):
            wb_wait(n_pos - NBUF + rb, rb)

    return k(tok_t, table)


def kernel(tokens, embedding_weight):
    n_batch, n_pos = tokens.shape
    tok_t = tokens.T.astype(jnp.int32)
    out4 = _gather_embed(tok_t, embedding_weight, n_pos, n_batch)
    out = (out4.reshape(n_pos, EMBED // 8, n_batch // TB, 8, TB)
           .transpose(2, 4, 0, 1, 3)
           .reshape(n_batch, n_pos, EMBED))
    return out


# speedup vs baseline: 1.1228x; 1.1228x over previous
import functools

import jax
import jax.numpy as jnp
from jax import lax
from jax.experimental import pallas as pl
from jax.experimental.pallas import tpu as pltpu
from jax.experimental.pallas import tpu_sc as plsc

VOCAB = 1000000
EMBED = 32
NUM_CORES = 2
NUM_SUBCORES = 16
NUM_WORKERS = NUM_CORES * NUM_SUBCORES
GB = 512
L = 16


@functools.partial(jax.jit, static_argnums=(2, 3))
def _gather_embed(tok_t, table, n_pos, n_batch):
    mesh = plsc.VectorSubcoreMesh(core_axis_name="c", subcore_axis_name="s")
    n_units = n_pos * (n_batch // GB)
    upw = n_units // NUM_WORKERS
    n_bg = n_batch // GB
    assert upw % 2 == 0

    @functools.partial(
        pl.kernel,
        mesh=mesh,
        out_type=jax.ShapeDtypeStruct((n_pos, EMBED // 8, n_batch * 8),
                                      jnp.float32),
        scratch_types=[pltpu.VMEM((GB,), jnp.int32)] * 2
          + [pltpu.VMEM((GB, EMBED), jnp.float32)] * 2
          + [pltpu.VMEM((GB * EMBED,), jnp.float32)] * 2
          + [pltpu.SemaphoreType.DMA] * 6,
        compiler_params=pltpu.CompilerParams(
            use_tc_tiling_on_sc=False, needs_layout_passes=False),
    )
    def k(tok_hbm, table_hbm, out_hbm, i0, i1, r0, r1, t0, t1,
          si0, si1, sg0, sg1, so0, so1):
        idx = (i0, i1)
        rows = (r0, r1)
        tbuf = (t0, t1)
        si = (si0, si1)
        sg = (sg0, sg1)
        so = (so0, so1)
        wid = lax.axis_index("s") * NUM_CORES + lax.axis_index("c")
        u0 = wid * upw

        iot = lax.iota(jnp.int32, L)
        bvecs = [iot + L * j for j in range(GB // L)]

        def idx_start(u, rb):
            pltpu.async_copy(
                tok_hbm.at[u // n_bg, pl.ds((u % n_bg) * GB, GB)],
                idx[rb], si[rb])

        def idx_wait(u, rb):
            pltpu.make_async_copy(
                tok_hbm.at[u // n_bg, pl.ds((u % n_bg) * GB, GB)],
                idx[rb], si[rb]).wait()

        def gather_start(rb):
            pltpu.async_copy(table_hbm.at[idx[rb]], rows[rb], sg[rb])

        def gather_wait(rb):
            pltpu.make_async_copy(
                table_hbm.at[idx[rb]], rows[rb], sg[rb]).wait()

        def wb_start(u, rb):
            for e8 in range(EMBED // 8):
                pltpu.async_copy(
                    tbuf[rb].at[pl.ds(e8 * 8 * GB, 8 * GB)],
                    out_hbm.at[u // n_bg, e8,
                               pl.ds((u % n_bg) * 8 * GB, 8 * GB)],
                    so[rb])

        def wb_wait(u, rb):
            for e8 in range(EMBED // 8):
                pltpu.make_async_copy(
                    tbuf[rb].at[pl.ds(e8 * 8 * GB, 8 * GB)],
                    out_hbm.at[u // n_bg, e8,
                               pl.ds((u % n_bg) * 8 * GB, 8 * GB)],
                    so[rb]).wait()

        def transpose(rb):
            def e_body(e, c):
                evec = jnp.full((L,), 0, jnp.int32) + e
                base_e = (e // 8) * (8 * GB) + (e % 8) * 128
                for j in range(GB // L):
                    v = plsc.load_gather(rows[rb], [bvecs[j], evec])
                    pos = base_e + (j // 8) * 1024 + (j % 8) * L
                    tbuf[rb][pl.ds(pos, L)] = v
                return c
            lax.fori_loop(0, EMBED, e_body, 0)

        idx_start(u0, 0)
        idx_wait(u0, 0)
        gather_start(0)
        idx_start(u0 + 1, 1)

        def body(kk, carry):
            for rb in range(2):
                i = 2 * kk + rb
                u = u0 + i
                if rb == 0:
                    idx_wait(u + 1, 1)
                    gather_start(1)
                else:
                    @pl.when(kk < upw // 2 - 1)
                    def _():
                        idx_wait(u + 1, 0)
                        gather_start(0)
                gather_wait(rb)
                @pl.when(kk < upw // 2 - 1)
                def _():
                    idx_start(u + 2, rb)
                @pl.when(kk > 0)
                def _():
                    wb_wait(u - 2, rb)
                transpose(rb)
                wb_start(u, rb)
            return carry

        lax.fori_loop(0, upw // 2, body, 0)
        wb_wait(u0 + upw - 2, 0)
        wb_wait(u0 + upw - 1, 1)

    return k(tok_t, table)


def kernel(tokens, embedding_weight):
    n_batch, n_pos = tokens.shape
    tok_t = tokens.T.astype(jnp.int32)
    out3 = _gather_embed(tok_t, embedding_weight, n_pos, n_batch)
    out = (out3.reshape(n_pos, EMBED // 8, n_batch // 128, 8, 128)
           .transpose(2, 4, 0, 1, 3)
           .reshape(n_batch, n_pos, EMBED))
    return out


# speedup vs baseline: 1.2395x; 1.1039x over previous
import functools

import jax
import jax.numpy as jnp
from jax import lax
from jax.experimental import pallas as pl
from jax.experimental.pallas import tpu as pltpu
from jax.experimental.pallas import tpu_sc as plsc

VOCAB = 1000000
EMBED = 32
NUM_CORES = 2
NUM_SUBCORES = 16
NUM_WORKERS = NUM_CORES * NUM_SUBCORES
L = 16
BB = 128
TQ = 4


@functools.partial(jax.jit, static_argnums=(2, 3))
def _gather_embed(tok, table, n_pos, n_batch):
    mesh = plsc.VectorSubcoreMesh(core_axis_name="c", subcore_axis_name="s")
    n_units = n_pos // TQ
    assert n_units % 2 == 0 and n_batch // BB == NUM_WORKERS
    GB = TQ * BB

    @functools.partial(
        pl.kernel,
        mesh=mesh,
        out_type=jax.ShapeDtypeStruct((n_pos, EMBED // 8, n_batch * 8),
                                      jnp.float32),
        scratch_types=[
            pltpu.VMEM((BB, n_pos), jnp.int32),
            pltpu.VMEM((n_pos * BB,), jnp.int32),
        ] + [pltpu.VMEM((GB, EMBED), jnp.float32)] * 2
          + [pltpu.VMEM((TQ, EMBED // 8, 8 * BB), jnp.float32)] * 2
          + [pltpu.SemaphoreType.DMA] * 4,
        compiler_params=pltpu.CompilerParams(
            use_tc_tiling_on_sc=False, needs_layout_passes=False),
    )
    def k(tok_hbm, table_hbm, out_hbm, slab, idxs, r0, r1, t0, t1,
          sg0, sg1, so0, so1):
        rows = (r0, r1)
        tbuf = (t0, t1)
        sg = (sg0, sg1)
        so = (so0, so1)
        wid = lax.axis_index("s") * NUM_CORES + lax.axis_index("c")
        b0 = wid * BB

        iot = lax.iota(jnp.int32, L)
        bvecs = [iot + L * j for j in range(BB // L)]

        pltpu.sync_copy(tok_hbm.at[pl.ds(b0, BB), :], slab)

        def slab_body(i, c):
            vs = []
            for jj in range(2 * BB // L):
                tvec = jnp.full((L,), 0, jnp.int32) + (2 * i + jj // 8)
                vs.append(plsc.load_gather(slab, [bvecs[jj % 8], tvec]))
            for jj in range(2 * BB // L):
                idxs[pl.ds(i * 2 * BB + jj * L, L)] = vs[jj]
            return c
        lax.fori_loop(0, n_pos // 2, slab_body, 0)

        def gather_start(u, rb):
            pltpu.async_copy(table_hbm.at[idxs.at[pl.ds(u * GB, GB)]],
                             rows[rb], sg[rb])

        def gather_wait(u, rb):
            pltpu.make_async_copy(table_hbm.at[idxs.at[pl.ds(u * GB, GB)]],
                                  rows[rb], sg[rb]).wait()

        def wb_start(u, rb):
            for tl in range(TQ):
                pltpu.async_copy(
                    tbuf[rb].at[tl],
                    out_hbm.at[u * TQ + tl, :, pl.ds(b0 * 8, 8 * BB)],
                    so[rb])

        def wb_wait(u, rb):
            for tl in range(TQ):
                pltpu.make_async_copy(
                    tbuf[rb].at[tl],
                    out_hbm.at[u * TQ + tl, :, pl.ds(b0 * 8, 8 * BB)],
                    so[rb]).wait()

        def transpose(rb):
            tb = tbuf[rb]
            def e_body(e, c):
                evec = jnp.full((L,), 0, jnp.int32) + e
                vs = []
                for j in range(GB // L):
                    vs.append(plsc.load_gather(rows[rb], [iot + L * j, evec]))
                for j in range(GB // L):
                    tb[j // 8, e // 8, pl.ds((e % 8) * BB + (j % 8) * L, L)] \
                        = vs[j]
                return c
            lax.fori_loop(0, EMBED, e_body, 0)

        gather_start(0, 0)

        def body(kk, carry):
            for rb in range(2):
                i = 2 * kk + rb
                if rb == 0:
                    gather_start(i + 1, 1)
                else:
                    @pl.when(kk < n_units // 2 - 1)
                    def _():
                        gather_start(i + 1, 0)
                gather_wait(i, rb)
                @pl.when(kk > 0)
                def _():
                    wb_wait(i - 2, rb)
                transpose(rb)
                wb_start(i, rb)
            return carry

        lax.fori_loop(0, n_units // 2, body, 0)
        wb_wait(n_units - 2, 0)
        wb_wait(n_units - 1, 1)

    return k(tok, table)


def kernel(tokens, embedding_weight):
    n_batch, n_pos = tokens.shape
    out3 = _gather_embed(tokens.astype(jnp.int32), embedding_weight,
                         n_pos, n_batch)
    out = (out3.reshape(n_pos, EMBED // 8, n_batch // 128, 8, 128)
           .transpose(2, 4, 0, 1, 3)
           .reshape(n_batch, n_pos, EMBED))
    return out
